# baseline (speedup 1.0000x reference)
import numpy as np
import jax
import jax.numpy as jnp
from jax import lax
from jax.experimental import pallas as pl
from jax.experimental.pallas import tpu as pltpu
from jax.experimental.pallas import tpu_sc as plsc

B = 524288
N_MIN, N_MAX, N_TABLES, MAX_TABLE_SIZE = 16, 512, 16, 131072
_b = np.exp((np.log(N_MAX) - np.log(N_MIN)) / (N_TABLES - 1))
N_L = [int(np.floor(N_MIN * _b ** i)) for i in range(N_TABLES)]
TABLE_SIZES = []
MAX_DIRECT = 0
for _i in range(N_TABLES):
    _ts = min(MAX_TABLE_SIZE, N_L[_i] * N_L[_i])
    if _ts == N_L[_i] * N_L[_i]:
        MAX_DIRECT = _i
        _ts = (N_L[_i] + 1) * (N_L[_i] + 1)
    TABLE_SIZES.append(_ts)
HASH1 = np.int32(265443576)

NC, NS = 2, 16
NW = NC * NS
PW = B // NW
C = 64
NCHUNK = PW // C
NG = C // 16
XBLK = 8
XW = 2 * C * XBLK

RESIDENT = [l for l in range(N_TABLES) if TABLE_SIZES[l] <= 4300]
STREAMED = [l for l in range(N_TABLES) if l not in RESIDENT]
NSTREAM = len(STREAMED)

_i32 = jnp.int32
_f32 = jnp.float32


def _fracs(l, xf, yf):
    n = jnp.float32(N_L[l])
    ux = xf * n
    uy = yf * n
    ix = ux.astype(_i32)
    iy = uy.astype(_i32)
    fx = ux - ix.astype(_f32)
    fy = uy - iy.astype(_f32)
    return ix, iy, fx, fy


def _corner_rows(l, ix, iy):
    if l <= MAX_DIRECT:
        nl = jnp.int32(N_L[l])
        i00 = iy * nl + ix
        i10 = i00 + 1
        i01 = i00 + nl
        i11 = i01 + 1
    else:
        m = jnp.int32(TABLE_SIZES[l] - 1)
        hy0 = iy * HASH1
        hy1 = hy0 + HASH1
        i00 = (ix ^ hy0) & m
        i10 = ((ix + 1) ^ hy0) & m
        i01 = (ix ^ hy1) & m
        i11 = ((ix + 1) ^ hy1) & m
    return i00, i10, i01, i11


def _lerp(a, b, t):
    return a + (b - a) * t


def _blend(v00, v10, v01, v11, fx, fy):
    return _lerp(_lerp(v00, v10, fx), _lerp(v01, v11, fx), fy)


def _body(x_hbm, *rest):
    grids = rest[:N_TABLES]
    out_hbm = rest[N_TABLES]
    sc = list(rest[N_TABLES + 1:])
    tbls = sc[:len(RESIDENT)]
    sc = sc[len(RESIDENT):]
    x_ab = sc[0:2]
    out_v = sc[2]
    idx_ab = [sc[3:3 + NSTREAM], sc[3 + NSTREAM:3 + 2 * NSTREAM]]
    gath_ab = [sc[3 + 2 * NSTREAM:3 + 3 * NSTREAM],
               sc[3 + 3 * NSTREAM:3 + 4 * NSTREAM]]
    spmems = sc[3 + 4 * NSTREAM:3 + 5 * NSTREAM]
    sem_x, sem_out = sc[3 + 5 * NSTREAM], sc[4 + 5 * NSTREAM]
    gsems = sc[5 + 5 * NSTREAM:]

    cid = lax.axis_index("c")
    sid = lax.axis_index("s")
    wid = sid * NC + cid
    iota = lax.iota(_i32, 16)
    out_stride = iota * 32

    for i, l in enumerate(RESIDENT):
        pltpu.sync_copy(grids[l], tbls[i])

    @pl.when(sid == 0)
    def _stage():
        for j, l in enumerate(STREAMED):
            pltpu.sync_copy(grids[l], spmems[j])

    plsc.subcore_barrier()

    xw0 = 2 * wid * PW

    def _read_xy(ref, off):
        xs, ys = [], []
        for g in range(NG):
            p2 = off + 32 * g + 2 * iota
            xs.append(plsc.load_gather(ref, [p2]))
            ys.append(plsc.load_gather(ref, [p2 + 1]))
        return tuple(xs + ys)

    def _phase_a(p, xs, ys):
        for j, l in enumerate(STREAMED):
            for g in range(NG):
                ix, iy, _, _ = _fracs(l, xs[g], ys[g])
                rows = _corner_rows(l, ix, iy)
                for c in range(4):
                    e0 = rows[c] + rows[c]
                    b0 = (2 * c) * C + g * 16
                    idx_ab[p][j][pl.ds(b0, 16)] = e0
                    idx_ab[p][j][pl.ds(b0 + C, 16)] = e0 + 1
            pltpu.async_copy(spmems[j].at[idx_ab[p][j]], gath_ab[p][j],
                             gsems[j])

    def _phase_b(p, l2j, xs, ys):
        for j, l in l2j:
            for g in range(NG):
                _, _, fx, fy = _fracs(l, xs[g], ys[g])
                v = [gath_ab[p][j][pl.ds(s * C + g * 16, 16)]
                     for s in range(8)]
                r0 = _blend(v[0], v[2], v[4], v[6], fx, fy)
                r1 = _blend(v[1], v[3], v[5], v[7], fx, fy)
                o0 = out_stride + (g * 16 * 32 + 2 * l)
                plsc.store_scatter(out_v, [o0], r0)
                plsc.store_scatter(out_v, [o0 + 1], r1)

    pltpu.async_copy(x_hbm.at[pl.ds(xw0, XW)], x_ab[0], sem_x)
    pltpu.make_async_copy(
        x_hbm.at[pl.ds(0, XW)], x_ab[0], sem_x).wait()
    pltpu.async_copy(x_hbm.at[pl.ds(xw0 + XW, XW)], x_ab[1], sem_x)
    xy0 = _read_xy(x_ab[0], 0)
    _phase_a(0, xy0[:NG], xy0[NG:])

    def chunk(ci, carry):
        xs = list(carry[:NG])
        ys = list(carry[NG:])
        base = wid * PW + ci * C
        parity = lax.rem(ci, jnp.int32(2))
        cn = ci + 1

        @pl.when((lax.rem(cn, jnp.int32(XBLK)) == 0) & (cn < NCHUNK))
        def _xblock():
            pltpu.make_async_copy(
                x_hbm.at[pl.ds(0, XW)], x_ab[0], sem_x).wait()
            nb = cn // XBLK + 1

            @pl.when(nb < NCHUNK // XBLK)
            def _fire():
                src = x_hbm.at[pl.ds(xw0 + nb * XW, XW)]

                @pl.when(lax.rem(nb, jnp.int32(2)) == 0)
                def _():
                    pltpu.async_copy(src, x_ab[0], sem_x)

                @pl.when(lax.rem(nb, jnp.int32(2)) == 1)
                def _():
                    pltpu.async_copy(src, x_ab[1], sem_x)

        cc = jnp.minimum(jnp.int32(cn), jnp.int32(NCHUNK - 1))
        off_n = lax.rem(cc, jnp.int32(XBLK)) * (2 * C)
        blk_par = lax.rem(cc // XBLK, jnp.int32(2))
        xy_a = _read_xy(x_ab[0], off_n)
        xy_b = _read_xy(x_ab[1], off_n)
        sel = blk_par == 0
        xy_n = tuple(jnp.where(sel, a, b) for a, b in zip(xy_a, xy_b))
        xs_n, ys_n = xy_n[:NG], xy_n[NG:]

        for j in range(NSTREAM):
            pltpu.make_async_copy(
                spmems[j].at[idx_ab[0][j]], gath_ab[0][j], gsems[j]).wait()

        @pl.when(ci < NCHUNK - 1)
        def _next_a():
            @pl.when(parity == 0)
            def _():
                _phase_a(1, xs_n, ys_n)

            @pl.when(parity == 1)
            def _():
                _phase_a(0, xs_n, ys_n)

        @pl.when(ci > 0)
        def _drain_out():
            pltpu.make_async_copy(
                out_v, out_hbm.at[pl.ds(0, 32 * C)], sem_out).wait()

        l2j = list(enumerate(STREAMED))

        @pl.when(parity == 0)
        def _b0():
            _phase_b(0, l2j, xs, ys)

        @pl.when(parity == 1)
        def _b1():
            _phase_b(1, l2j, xs, ys)

        for i, l in enumerate(RESIDENT):
            for g in range(NG):
                ix, iy, fx, fy = _fracs(l, xs[g], ys[g])
                i00, i10, i01, i11 = _corner_rows(l, ix, iy)
                e00, e10 = i00 + i00, i10 + i10
                e01, e11 = i01 + i01, i11 + i11
                r0 = _blend(
                    plsc.load_gather(tbls[i], [e00]),
                    plsc.load_gather(tbls[i], [e10]),
                    plsc.load_gather(tbls[i], [e01]),
                    plsc.load_gather(tbls[i], [e11]),
                    fx, fy)
                r1 = _blend(
                    plsc.load_gather(tbls[i], [e00 + 1]),
                    plsc.load_gather(tbls[i], [e10 + 1]),
                    plsc.load_gather(tbls[i], [e01 + 1]),
                    plsc.load_gather(tbls[i], [e11 + 1]),
                    fx, fy)
                o0 = out_stride + (g * 16 * 32 + 2 * l)
                plsc.store_scatter(out_v, [o0], r0)
                plsc.store_scatter(out_v, [o0 + 1], r1)

        pltpu.async_copy(out_v, out_hbm.at[pl.ds(32 * base, 32 * C)], sem_out)
        return xy_n

    lax.fori_loop(0, NCHUNK, chunk, xy0)
    pltpu.make_async_copy(out_v, out_hbm.at[pl.ds(0, 32 * C)], sem_out).wait()


def _build():
    scratch = [pltpu.VMEM((2 * TABLE_SIZES[l],), _f32) for l in RESIDENT]
    scratch += [
        pltpu.VMEM((2 * C * XBLK,), _f32),
        pltpu.VMEM((2 * C * XBLK,), _f32),
        pltpu.VMEM((32 * C,), _f32),
    ]
    scratch += [pltpu.VMEM((8 * C,), _i32) for _ in STREAMED]
    scratch += [pltpu.VMEM((8 * C,), _i32) for _ in STREAMED]
    scratch += [pltpu.VMEM((8 * C,), _f32) for _ in STREAMED]
    scratch += [pltpu.VMEM((8 * C,), _f32) for _ in STREAMED]
    scratch += [pltpu.VMEM_SHARED((2 * TABLE_SIZES[l],), _f32)
                for l in STREAMED]
    scratch += [pltpu.SemaphoreType.DMA, pltpu.SemaphoreType.DMA]
    scratch += [pltpu.SemaphoreType.DMA for _ in STREAMED]
    mesh = plsc.VectorSubcoreMesh(core_axis_name="c", subcore_axis_name="s")
    return pl.kernel(
        _body,
        out_type=jax.ShapeDtypeStruct((B * 32,), _f32),
        mesh=mesh,
        scratch_types=scratch,
        compiler_params=pltpu.CompilerParams(needs_layout_passes=False),
    )


_encode_sc = _build()


@jax.jit
def kernel(x, grid0, grid1, grid2, grid3, grid4, grid5, grid6, grid7,
           grid8, grid9, grid10, grid11, grid12, grid13, grid14, grid15):
    grids = [grid0, grid1, grid2, grid3, grid4, grid5, grid6, grid7,
             grid8, grid9, grid10, grid11, grid12, grid13, grid14, grid15]
    flat = _encode_sc(x.reshape(-1), *[g.reshape(-1) for g in grids])
    return flat.reshape(B, 32)


# speedup vs baseline: 1.0149x; 1.0149x over previous
import numpy as np
import jax
import jax.numpy as jnp
from jax import lax
from jax.experimental import pallas as pl
from jax.experimental.pallas import tpu as pltpu
from jax.experimental.pallas import tpu_sc as plsc

B = 524288
N_MIN, N_MAX, N_TABLES, MAX_TABLE_SIZE = 16, 512, 16, 131072
_b = np.exp((np.log(N_MAX) - np.log(N_MIN)) / (N_TABLES - 1))
N_L = [int(np.floor(N_MIN * _b ** i)) for i in range(N_TABLES)]
TABLE_SIZES = []
MAX_DIRECT = 0
for _i in range(N_TABLES):
    _ts = min(MAX_TABLE_SIZE, N_L[_i] * N_L[_i])
    if _ts == N_L[_i] * N_L[_i]:
        MAX_DIRECT = _i
        _ts = (N_L[_i] + 1) * (N_L[_i] + 1)
    TABLE_SIZES.append(_ts)
HASH1 = np.int32(265443576)

NC, NS = 2, 16
NW = NC * NS
PW = B // NW
C = 64
NCHUNK = PW // C
NG = C // 16
XBLK = 8
XW = 2 * C * XBLK

RESIDENT = [l for l in range(N_TABLES) if TABLE_SIZES[l] <= 4300]
STREAMED = [l for l in range(N_TABLES) if l not in RESIDENT]
NSTREAM = len(STREAMED)

_i32 = jnp.int32
_f32 = jnp.float32


def _fracs(l, xf, yf):
    n = jnp.float32(N_L[l])
    ux = xf * n
    uy = yf * n
    ix = ux.astype(_i32)
    iy = uy.astype(_i32)
    fx = ux - ix.astype(_f32)
    fy = uy - iy.astype(_f32)
    return ix, iy, fx, fy


def _corner_rows(l, ix, iy):
    if l <= MAX_DIRECT:
        nl = jnp.int32(N_L[l])
        i00 = iy * nl + ix
        i10 = i00 + 1
        i01 = i00 + nl
        i11 = i01 + 1
    else:
        m = jnp.int32(TABLE_SIZES[l] - 1)
        hy0 = iy * HASH1
        hy1 = hy0 + HASH1
        i00 = (ix ^ hy0) & m
        i10 = ((ix + 1) ^ hy0) & m
        i01 = (ix ^ hy1) & m
        i11 = ((ix + 1) ^ hy1) & m
    return i00, i10, i01, i11


def _lerp(a, b, t):
    return a + (b - a) * t


def _blend(v00, v10, v01, v11, fx, fy):
    return _lerp(_lerp(v00, v10, fx), _lerp(v01, v11, fx), fy)


def _body(x_hbm, *rest):
    grids = rest[:N_TABLES]
    out_hbm = rest[N_TABLES]
    sc = list(rest[N_TABLES + 1:])
    tbls = sc[:len(RESIDENT)]
    sc = sc[len(RESIDENT):]
    x_ab = sc[0:2]
    out_v = sc[2]
    out_c = sc.pop(3 + 5 * len(STREAMED))
    idx_ab = [sc[3:3 + NSTREAM], sc[3 + NSTREAM:3 + 2 * NSTREAM]]
    gath_ab = [sc[3 + 2 * NSTREAM:3 + 3 * NSTREAM],
               sc[3 + 3 * NSTREAM:3 + 4 * NSTREAM]]
    spmems = sc[3 + 4 * NSTREAM:3 + 5 * NSTREAM]
    sem_x, sem_out = sc[3 + 5 * NSTREAM], sc[4 + 5 * NSTREAM]
    gsems = sc[5 + 5 * NSTREAM:]

    cid = lax.axis_index("c")
    sid = lax.axis_index("s")
    wid = sid * NC + cid
    iota = lax.iota(_i32, 16)
    out_stride = iota * 33

    for i, l in enumerate(RESIDENT):
        pltpu.sync_copy(grids[l], tbls[i])

    @pl.when(sid == 0)
    def _stage():
        for j, l in enumerate(STREAMED):
            pltpu.sync_copy(grids[l], spmems[j])

    plsc.subcore_barrier()

    xw0 = 2 * wid * PW

    def _read_xy(ref, off):
        xs, ys = [], []
        for g in range(NG):
            p2 = off + 32 * g + 2 * iota
            xs.append(plsc.load_gather(ref, [p2]))
            ys.append(plsc.load_gather(ref, [p2 + 1]))
        return tuple(xs + ys)

    def _phase_a(p, xs, ys):
        for j, l in enumerate(STREAMED):
            for g in range(NG):
                ix, iy, _, _ = _fracs(l, xs[g], ys[g])
                rows = _corner_rows(l, ix, iy)
                for c in range(4):
                    e0 = rows[c] + rows[c]
                    b0 = (2 * c) * C + g * 16
                    idx_ab[p][j][pl.ds(b0, 16)] = e0
                    idx_ab[p][j][pl.ds(b0 + C, 16)] = e0 + 1
            pltpu.async_copy(spmems[j].at[idx_ab[p][j]], gath_ab[p][j],
                             gsems[j])

    def _phase_b(p, l2j, xs, ys):
        for j, l in l2j:
            for g in range(NG):
                _, _, fx, fy = _fracs(l, xs[g], ys[g])
                v = [gath_ab[p][j][pl.ds(s * C + g * 16, 16)]
                     for s in range(8)]
                r0 = _blend(v[0], v[2], v[4], v[6], fx, fy)
                r1 = _blend(v[1], v[3], v[5], v[7], fx, fy)
                o0 = out_stride + (g * 16 * 33 + 2 * l)
                plsc.store_scatter(out_v, [o0], r0)
                plsc.store_scatter(out_v, [o0 + 1], r1)

    pltpu.async_copy(x_hbm.at[pl.ds(xw0, XW)], x_ab[0], sem_x)
    pltpu.make_async_copy(
        x_hbm.at[pl.ds(0, XW)], x_ab[0], sem_x).wait()
    pltpu.async_copy(x_hbm.at[pl.ds(xw0 + XW, XW)], x_ab[1], sem_x)
    xy0 = _read_xy(x_ab[0], 0)
    _phase_a(0, xy0[:NG], xy0[NG:])

    def chunk(ci, carry):
        xs = list(carry[:NG])
        ys = list(carry[NG:])
        base = wid * PW + ci * C
        parity = lax.rem(ci, jnp.int32(2))
        cn = ci + 1

        @pl.when((lax.rem(cn, jnp.int32(XBLK)) == 0) & (cn < NCHUNK))
        def _xblock():
            pltpu.make_async_copy(
                x_hbm.at[pl.ds(0, XW)], x_ab[0], sem_x).wait()
            nb = cn // XBLK + 1

            @pl.when(nb < NCHUNK // XBLK)
            def _fire():
                src = x_hbm.at[pl.ds(xw0 + nb * XW, XW)]

                @pl.when(lax.rem(nb, jnp.int32(2)) == 0)
                def _():
                    pltpu.async_copy(src, x_ab[0], sem_x)

                @pl.when(lax.rem(nb, jnp.int32(2)) == 1)
                def _():
                    pltpu.async_copy(src, x_ab[1], sem_x)

        cc = jnp.minimum(jnp.int32(cn), jnp.int32(NCHUNK - 1))
        off_n = lax.rem(cc, jnp.int32(XBLK)) * (2 * C)
        blk_par = lax.rem(cc // XBLK, jnp.int32(2))
        xy_a = _read_xy(x_ab[0], off_n)
        xy_b = _read_xy(x_ab[1], off_n)
        sel = blk_par == 0
        xy_n = tuple(jnp.where(sel, a, b) for a, b in zip(xy_a, xy_b))
        xs_n, ys_n = xy_n[:NG], xy_n[NG:]

        for j in range(NSTREAM):
            pltpu.make_async_copy(
                spmems[j].at[idx_ab[0][j]], gath_ab[0][j], gsems[j]).wait()

        @pl.when(ci < NCHUNK - 1)
        def _next_a():
            @pl.when(parity == 0)
            def _():
                _phase_a(1, xs_n, ys_n)

            @pl.when(parity == 1)
            def _():
                _phase_a(0, xs_n, ys_n)

        @pl.when(ci > 0)
        def _drain_out():
            pltpu.make_async_copy(
                out_c, out_hbm.at[pl.ds(0, 32 * C)], sem_out).wait()

        l2j = list(enumerate(STREAMED))

        @pl.when(parity == 0)
        def _b0():
            _phase_b(0, l2j, xs, ys)

        @pl.when(parity == 1)
        def _b1():
            _phase_b(1, l2j, xs, ys)

        for i, l in enumerate(RESIDENT):
            for g in range(NG):
                ix, iy, fx, fy = _fracs(l, xs[g], ys[g])
                i00, i10, i01, i11 = _corner_rows(l, ix, iy)
                e00, e10 = i00 + i00, i10 + i10
                e01, e11 = i01 + i01, i11 + i11
                r0 = _blend(
                    plsc.load_gather(tbls[i], [e00]),
                    plsc.load_gather(tbls[i], [e10]),
                    plsc.load_gather(tbls[i], [e01]),
                    plsc.load_gather(tbls[i], [e11]),
                    fx, fy)
                r1 = _blend(
                    plsc.load_gather(tbls[i], [e00 + 1]),
                    plsc.load_gather(tbls[i], [e10 + 1]),
                    plsc.load_gather(tbls[i], [e01 + 1]),
                    plsc.load_gather(tbls[i], [e11 + 1]),
                    fx, fy)
                o0 = out_stride + (g * 16 * 33 + 2 * l)
                plsc.store_scatter(out_v, [o0], r0)
                plsc.store_scatter(out_v, [o0 + 1], r1)

        for p in range(C):
            out_c[pl.ds(32 * p, 16)] = out_v[pl.ds(33 * p, 16)]
            out_c[pl.ds(32 * p + 16, 16)] = out_v[pl.ds(33 * p + 16, 16)]

        pltpu.async_copy(out_c, out_hbm.at[pl.ds(32 * base, 32 * C)], sem_out)
        return xy_n

    lax.fori_loop(0, NCHUNK, chunk, xy0)
    pltpu.make_async_copy(out_c, out_hbm.at[pl.ds(0, 32 * C)], sem_out).wait()


def _build():
    scratch = [pltpu.VMEM((2 * TABLE_SIZES[l],), _f32) for l in RESIDENT]
    scratch += [
        pltpu.VMEM((2 * C * XBLK,), _f32),
        pltpu.VMEM((2 * C * XBLK,), _f32),
        pltpu.VMEM((33 * C,), _f32),
    ]
    scratch += [pltpu.VMEM((8 * C,), _i32) for _ in STREAMED]
    scratch += [pltpu.VMEM((8 * C,), _i32) for _ in STREAMED]
    scratch += [pltpu.VMEM((8 * C,), _f32) for _ in STREAMED]
    scratch += [pltpu.VMEM((8 * C,), _f32) for _ in STREAMED]
    scratch += [pltpu.VMEM_SHARED((2 * TABLE_SIZES[l],), _f32)
                for l in STREAMED]
    scratch.insert(3 + len(RESIDENT) + 5 * len(STREAMED),
                   pltpu.VMEM((32 * C,), _f32))
    scratch += [pltpu.SemaphoreType.DMA, pltpu.SemaphoreType.DMA]
    scratch += [pltpu.SemaphoreType.DMA for _ in STREAMED]
    mesh = plsc.VectorSubcoreMesh(core_axis_name="c", subcore_axis_name="s")
    return pl.kernel(
        _body,
        out_type=jax.ShapeDtypeStruct((B * 32,), _f32),
        mesh=mesh,
        scratch_types=scratch,
        compiler_params=pltpu.CompilerParams(needs_layout_passes=False),
    )


_encode_sc = _build()


@jax.jit
def kernel(x, grid0, grid1, grid2, grid3, grid4, grid5, grid6, grid7,
           grid8, grid9, grid10, grid11, grid12, grid13, grid14, grid15):
    grids = [grid0, grid1, grid2, grid3, grid4, grid5, grid6, grid7,
             grid8, grid9, grid10, grid11, grid12, grid13, grid14, grid15]
    flat = _encode_sc(x.reshape(-1), *[g.reshape(-1) for g in grids])
    return flat.reshape(B, 32)
